# baseline (device time: 24695 ns/iter reference)
import jax
import jax.numpy as jnp
from jax import lax
from jax.experimental import pallas as pl
from jax.experimental.pallas import tpu as pltpu

N_DEV = 8
SUB = 16
LANE = 128
CHUNKS = 4
TR = SUB // CHUNKS


def kernel(x):
    m, n = x.shape
    assert m == SUB * LANE
    rows_per = m // CHUNKS

    def body(x_ref, out_ref, m_comm, s_comm,
             m_send_sems, m_recv_sems, s_send_sems, s_recv_sems):
        me = lax.axis_index("i")

        barrier_sem = pltpu.get_barrier_semaphore()
        for off in range(1, N_DEV):
            peer = (me + off) % N_DEV
            pl.semaphore_signal(
                barrier_sem, inc=1,
                device_id=(peer,), device_id_type=pl.DeviceIdType.MESH,
            )
        pl.semaphore_wait(barrier_sem, N_DEV - 1)

        def chunk3(ref, c):
            return ref[pl.ds(c * rows_per, rows_per), :].reshape(TR, LANE, n)

        m_locs = []
        for c in range(CHUNKS):
            m_locs.append(jnp.max(chunk3(x_ref, c), axis=2))
        m_loc = jnp.concatenate(m_locs, axis=0)
        m_comm[me] = m_loc
        m_rdmas = []
        for off in range(1, N_DEV):
            peer = (me + off) % N_DEV
            rdma = pltpu.make_async_remote_copy(
                src_ref=m_comm.at[me],
                dst_ref=m_comm.at[me],
                send_sem=m_send_sems.at[off - 1],
                recv_sem=m_recv_sems.at[off - 1],
                device_id=(peer,),
                device_id_type=pl.DeviceIdType.MESH,
            )
            rdma.start()
            m_rdmas.append(rdma)

        s_locs = []
        for c in range(CHUNKS):
            e = jnp.exp(chunk3(x_ref, c) - m_locs[c][:, :, None])
            s_locs.append(jnp.sum(e, axis=2))
        s_comm[me] = jnp.concatenate(s_locs, axis=0)

        s_rdmas = []
        for off in range(1, N_DEV):
            peer = (me + off) % N_DEV
            rdma = pltpu.make_async_remote_copy(
                src_ref=s_comm.at[me],
                dst_ref=s_comm.at[me],
                send_sem=s_send_sems.at[off - 1],
                recv_sem=s_recv_sems.at[off - 1],
                device_id=(peer,),
                device_id_type=pl.DeviceIdType.MESH,
            )
            rdma.start()
            s_rdmas.append(rdma)

        for off in range(1, N_DEV):
            src = (me - off) % N_DEV
            recv = pltpu.make_async_remote_copy(
                src_ref=m_comm.at[me],
                dst_ref=m_comm.at[src],
                send_sem=m_send_sems.at[off - 1],
                recv_sem=m_recv_sems.at[off - 1],
                device_id=(src,),
                device_id_type=pl.DeviceIdType.MESH,
            )
            recv.wait_recv()
        m_all = m_comm[:, :, :]
        m_g = jnp.max(m_all, axis=0)

        for off in range(1, N_DEV):
            src = (me - off) % N_DEV
            recv = pltpu.make_async_remote_copy(
                src_ref=s_comm.at[me],
                dst_ref=s_comm.at[src],
                send_sem=s_send_sems.at[off - 1],
                recv_sem=s_recv_sems.at[off - 1],
                device_id=(src,),
                device_id_type=pl.DeviceIdType.MESH,
            )
            recv.wait_recv()
        s_all = s_comm[:, :, :]
        s_g = jnp.sum(s_all * jnp.exp(m_all - m_g[None]), axis=0)
        r = 1.0 / s_g

        for c in range(CHUNKS):
            mg_c = m_g[c * TR:(c + 1) * TR][:, :, None]
            r_c = r[c * TR:(c + 1) * TR][:, :, None]
            out_c = jnp.exp(chunk3(x_ref, c) - mg_c) * r_c
            out_ref[pl.ds(c * rows_per, rows_per), :] = out_c.reshape(
                rows_per, n
            )

        for rdma in m_rdmas + s_rdmas:
            rdma.wait_send()

    return pl.pallas_call(
        body,
        out_shape=jax.ShapeDtypeStruct((m, n), jnp.float32),
        in_specs=[pl.BlockSpec(memory_space=pltpu.VMEM)],
        out_specs=pl.BlockSpec(memory_space=pltpu.VMEM),
        scratch_shapes=[
            pltpu.VMEM((N_DEV, SUB, LANE), jnp.float32),
            pltpu.VMEM((N_DEV, SUB, LANE), jnp.float32),
            pltpu.SemaphoreType.DMA((N_DEV - 1,)),
            pltpu.SemaphoreType.DMA((N_DEV - 1,)),
            pltpu.SemaphoreType.DMA((N_DEV - 1,)),
            pltpu.SemaphoreType.DMA((N_DEV - 1,)),
        ],
        compiler_params=pltpu.CompilerParams(collective_id=0),
    )(x)
